# baseline (speedup 1.0000x reference)
import functools

import jax
import jax.numpy as jnp
from jax import lax
from jax.experimental import pallas as pl
from jax.experimental.pallas import tpu as pltpu
from jax.experimental.pallas import tpu_sc as plsc

NC, NS = 1, 16
NW = NC * NS
B, L, D = 4096, 50, 64
N_ROWS = B * L
ROWS_PER_W = N_ROWS // NW
CHUNK = 128
N_CHUNKS = ROWS_PER_W // CHUNK
N_PAS, T_RANGE = 8, 2048
N_COMBO = N_PAS * T_RANGE
EPS = 1e-5


def _combo_body(pas_ref, mjd_ref, out_ref):
    mjd = mjd_ref[...]
    for p in range(N_PAS):
        out_ref[p * T_RANGE:(p + 1) * T_RANGE, :] = mjd + pas_ref[p:p + 1, :]


def _build_combo(passend_table, mjd_table):
    return pl.pallas_call(
        _combo_body,
        out_shape=jax.ShapeDtypeStruct((N_COMBO, D), jnp.float32),
    )(passend_table, mjd_table)


def _rsqrt2(x):
    i = plsc.bitcast(x, jnp.int32)
    i = jnp.int32(0x5F3759DF) - (i >> 1)
    y = plsc.bitcast(i, jnp.float32)
    for _ in range(2):
        y = y * (1.5 - 0.5 * x * y * y)
    return y


_BCAST_DNUMS = lax.GatherDimensionNumbers(
    offset_dims=(), collapsed_slice_dims=(0,), start_index_map=(0,))


def _lane_bcast(vec, lane):
    idx = jnp.full((16, 1), lane, jnp.int32)
    return lax.gather(vec, idx, _BCAST_DNUMS, (1,),
                      mode=lax.GatherScatterMode.PROMISE_IN_BOUNDS)


def _sc_body(x_hbm, cidx_hbm, tok_hbm, combo_hbm, gamma_hbm, beta_hbm,
             out_hbm, xidx_all, cidx_all,
             trows0, trows1, crows0, crows1, e0, e1,
             gamma_v, beta_v,
             sem_t0, sem_t1, sem_c0, sem_c1, sem_o0, sem_o1):
    wid = lax.axis_index("s") * NC + lax.axis_index("c")
    base = wid * ROWS_PER_W

    trows = (trows0, trows1)
    crows = (crows0, crows1)
    ebuf = (e0, e1)
    sem_t = (sem_t0, sem_t1)
    sem_c = (sem_c0, sem_c1)
    sem_o = (sem_o0, sem_o1)

    pltpu.sync_copy(gamma_hbm, gamma_v)
    pltpu.sync_copy(beta_hbm, beta_v)
    gammas = [gamma_v[pl.ds(k * 16, 16)] for k in range(D // 16)]
    betas = [beta_v[pl.ds(k * 16, 16)] for k in range(D // 16)]

    pltpu.sync_copy(x_hbm.at[pl.ds(wid * N_CHUNKS, N_CHUNKS)], xidx_all)
    pltpu.sync_copy(cidx_hbm.at[pl.ds(wid * N_CHUNKS, N_CHUNKS)], cidx_all)

    def gathers(b, ci):
        return (
            pltpu.make_async_copy(tok_hbm.at[xidx_all.at[ci]], trows[b],
                                  sem_t[b]),
            pltpu.make_async_copy(combo_hbm.at[cidx_all.at[ci]], crows[b],
                                  sem_c[b]),
        )

    def out_copy(b, ci):
        row0 = base + ci * CHUNK
        return pltpu.make_async_copy(ebuf[b], out_hbm.at[pl.ds(row0, CHUNK)],
                                     sem_o[b])

    def compute(t_ref, c_ref, e_ref):
        def group_body(g, gcarry):
            for r in range(16):
                row = g * 16 + r
                v = [t_ref[row, pl.ds(k * 16, 16)]
                     + c_ref[row, pl.ds(k * 16, 16)]
                     for k in range(D // 16)]
                svec = (v[0] + v[1]) + (v[2] + v[3])
                qvec = ((v[0] * v[0] + v[1] * v[1])
                        + (v[2] * v[2] + v[3] * v[3]))
                stot = _lane_bcast(plsc.cumsum(svec), 15)
                qtot = _lane_bcast(plsc.cumsum(qvec), 15)
                mean = stot * (1.0 / D)
                var = qtot * (1.0 / D) - mean * mean
                rstd = _rsqrt2(var + EPS)
                for k in range(D // 16):
                    e_ref[row, pl.ds(k * 16, 16)] = (
                        (v[k] - mean) * (rstd * gammas[k]) + betas[k])
            return gcarry

        lax.fori_loop(0, CHUNK // 16, group_body, 0)

    for b in range(2):
        for cp in gathers(b, b):
            cp.start()

    def pair_body(j, carry):
        for b in range(2):
            ci = 2 * j + b
            for cp in gathers(b, ci):
                cp.wait()

            @pl.when(ci >= 2)
            def _():
                out_copy(b, ci - 2).wait()

            compute(trows[b], crows[b], ebuf[b])

            @pl.when(ci < N_CHUNKS - 2)
            def _():
                for cp in gathers(b, ci + 2):
                    cp.start()

            out_copy(b, ci).start()
        return carry

    lax.fori_loop(0, N_CHUNKS // 2, pair_body, 0)

    for b in range(2):
        out_copy(b, N_CHUNKS - 2 + b).wait()


_sc_kernel = functools.partial(
    pl.kernel,
    out_type=jax.ShapeDtypeStruct((N_ROWS, D), jnp.float32),
    mesh=plsc.VectorSubcoreMesh(
        core_axis_name="c", subcore_axis_name="s",
        num_cores=NC, num_subcores=NS),
    scratch_types=[
        pltpu.VMEM((N_CHUNKS, CHUNK), jnp.int32),
        pltpu.VMEM((N_CHUNKS, CHUNK), jnp.int32),
        pltpu.VMEM((CHUNK, D), jnp.float32),
        pltpu.VMEM((CHUNK, D), jnp.float32),
        pltpu.VMEM((CHUNK, D), jnp.float32),
        pltpu.VMEM((CHUNK, D), jnp.float32),
        pltpu.VMEM((CHUNK, D), jnp.float32),
        pltpu.VMEM((CHUNK, D), jnp.float32),
        pltpu.VMEM((D,), jnp.float32),
        pltpu.VMEM((D,), jnp.float32),
        pltpu.SemaphoreType.DMA,
        pltpu.SemaphoreType.DMA,
        pltpu.SemaphoreType.DMA,
        pltpu.SemaphoreType.DMA,
        pltpu.SemaphoreType.DMA,
        pltpu.SemaphoreType.DMA,
    ],
    compiler_params=pltpu.CompilerParams(
        needs_layout_passes=False, use_tc_tiling_on_sc=False),
)(_sc_body)


def kernel(x, mjd, passend, tok_table, passend_table, mjd_table,
           ln_gamma, ln_beta):
    combo = _build_combo(passend_table, mjd_table)
    x2 = x.reshape(N_ROWS // CHUNK, CHUNK).astype(jnp.int32)
    cidx2 = (passend.reshape(N_ROWS // CHUNK, CHUNK).astype(jnp.int32)
             * T_RANGE
             + mjd.reshape(N_ROWS // CHUNK, CHUNK).astype(jnp.int32))
    out = _sc_kernel(x2, cidx2, tok_table, combo, ln_gamma, ln_beta)
    return out.reshape(B, L, D)


# speedup vs baseline: 1.0983x; 1.0983x over previous
import functools

import jax
import jax.numpy as jnp
from jax import lax
from jax.experimental import pallas as pl
from jax.experimental.pallas import tpu as pltpu
from jax.experimental.pallas import tpu_sc as plsc

NC, NS = 2, 16
NW = NC * NS
B, L, D = 4096, 50, 64
N_ROWS = B * L
ROWS_PER_W = N_ROWS // NW
CHUNK = 128
N_CHUNKS = ROWS_PER_W // CHUNK
N_PAS, T_RANGE = 8, 2048
N_COMBO = N_PAS * T_RANGE
EPS = 1e-5


def _combo_body(pas_ref, mjd_ref, out_ref):
    mjd = mjd_ref[...]
    for p in range(N_PAS):
        out_ref[p * T_RANGE:(p + 1) * T_RANGE, :] = mjd + pas_ref[p:p + 1, :]


def _build_combo(passend_table, mjd_table):
    return pl.pallas_call(
        _combo_body,
        out_shape=jax.ShapeDtypeStruct((N_COMBO, D), jnp.float32),
    )(passend_table, mjd_table)


def _rsqrt2(x):
    i = plsc.bitcast(x, jnp.int32)
    i = jnp.int32(0x5F3759DF) - (i >> 1)
    y = plsc.bitcast(i, jnp.float32)
    for _ in range(2):
        y = y * (1.5 - 0.5 * x * y * y)
    return y


_BCAST_DNUMS = lax.GatherDimensionNumbers(
    offset_dims=(), collapsed_slice_dims=(0,), start_index_map=(0,))


def _lane_bcast(vec, lane):
    idx = jnp.full((16, 1), lane, jnp.int32)
    return lax.gather(vec, idx, _BCAST_DNUMS, (1,),
                      mode=lax.GatherScatterMode.PROMISE_IN_BOUNDS)


def _sc_body(x_hbm, cidx_hbm, tok_hbm, combo_hbm, gamma_hbm, beta_hbm,
             out_hbm, xidx_all, cidx_all,
             trows0, trows1, crows0, crows1, e0, e1,
             gamma_v, beta_v,
             sem_t0, sem_t1, sem_c0, sem_c1, sem_o0, sem_o1):
    wid = lax.axis_index("s") * NC + lax.axis_index("c")
    base = wid * ROWS_PER_W

    trows = (trows0, trows1)
    crows = (crows0, crows1)
    ebuf = (e0, e1)
    sem_t = (sem_t0, sem_t1)
    sem_c = (sem_c0, sem_c1)
    sem_o = (sem_o0, sem_o1)

    pltpu.sync_copy(gamma_hbm, gamma_v)
    pltpu.sync_copy(beta_hbm, beta_v)
    gammas = [gamma_v[pl.ds(k * 16, 16)] for k in range(D // 16)]
    betas = [beta_v[pl.ds(k * 16, 16)] for k in range(D // 16)]

    pltpu.sync_copy(x_hbm.at[pl.ds(wid * N_CHUNKS, N_CHUNKS)], xidx_all)
    pltpu.sync_copy(cidx_hbm.at[pl.ds(wid * N_CHUNKS, N_CHUNKS)], cidx_all)

    def gathers(b, ci):
        return (
            pltpu.make_async_copy(tok_hbm.at[xidx_all.at[ci]], trows[b],
                                  sem_t[b]),
            pltpu.make_async_copy(combo_hbm.at[cidx_all.at[ci]], crows[b],
                                  sem_c[b]),
        )

    def out_copy(b, ci):
        row0 = base + ci * CHUNK
        return pltpu.make_async_copy(ebuf[b], out_hbm.at[pl.ds(row0, CHUNK)],
                                     sem_o[b])

    def compute(t_ref, c_ref, e_ref):
        def group_body(g, gcarry):
            for r in range(16):
                row = g * 16 + r
                v = [t_ref[row, pl.ds(k * 16, 16)]
                     + c_ref[row, pl.ds(k * 16, 16)]
                     for k in range(D // 16)]
                svec = (v[0] + v[1]) + (v[2] + v[3])
                qvec = ((v[0] * v[0] + v[1] * v[1])
                        + (v[2] * v[2] + v[3] * v[3]))
                stot = _lane_bcast(plsc.cumsum(svec), 15)
                qtot = _lane_bcast(plsc.cumsum(qvec), 15)
                mean = stot * (1.0 / D)
                var = qtot * (1.0 / D) - mean * mean
                rstd = _rsqrt2(var + EPS)
                for k in range(D // 16):
                    e_ref[row, pl.ds(k * 16, 16)] = (
                        (v[k] - mean) * (rstd * gammas[k]) + betas[k])
            return gcarry

        lax.fori_loop(0, CHUNK // 16, group_body, 0)

    for b in range(2):
        for cp in gathers(b, b):
            cp.start()

    def pair_body(j, carry):
        for b in range(2):
            ci = 2 * j + b
            for cp in gathers(b, ci):
                cp.wait()

            @pl.when(ci >= 2)
            def _():
                out_copy(b, ci - 2).wait()

            compute(trows[b], crows[b], ebuf[b])

            @pl.when(ci < N_CHUNKS - 2)
            def _():
                for cp in gathers(b, ci + 2):
                    cp.start()

            out_copy(b, ci).start()
        return carry

    lax.fori_loop(0, N_CHUNKS // 2, pair_body, 0)

    for b in range(2):
        out_copy(b, N_CHUNKS - 2 + b).wait()


_sc_kernel = functools.partial(
    pl.kernel,
    out_type=jax.ShapeDtypeStruct((N_ROWS, D), jnp.float32),
    mesh=plsc.VectorSubcoreMesh(
        core_axis_name="c", subcore_axis_name="s",
        num_cores=NC, num_subcores=NS),
    scratch_types=[
        pltpu.VMEM((N_CHUNKS, CHUNK), jnp.int32),
        pltpu.VMEM((N_CHUNKS, CHUNK), jnp.int32),
        pltpu.VMEM((CHUNK, D), jnp.float32),
        pltpu.VMEM((CHUNK, D), jnp.float32),
        pltpu.VMEM((CHUNK, D), jnp.float32),
        pltpu.VMEM((CHUNK, D), jnp.float32),
        pltpu.VMEM((CHUNK, D), jnp.float32),
        pltpu.VMEM((CHUNK, D), jnp.float32),
        pltpu.VMEM((D,), jnp.float32),
        pltpu.VMEM((D,), jnp.float32),
        pltpu.SemaphoreType.DMA,
        pltpu.SemaphoreType.DMA,
        pltpu.SemaphoreType.DMA,
        pltpu.SemaphoreType.DMA,
        pltpu.SemaphoreType.DMA,
        pltpu.SemaphoreType.DMA,
    ],
    compiler_params=pltpu.CompilerParams(
        needs_layout_passes=False, use_tc_tiling_on_sc=False),
)(_sc_body)


def kernel(x, mjd, passend, tok_table, passend_table, mjd_table,
           ln_gamma, ln_beta):
    combo = _build_combo(passend_table, mjd_table)
    x2 = x.reshape(N_ROWS // CHUNK, CHUNK).astype(jnp.int32)
    cidx2 = (passend.reshape(N_ROWS // CHUNK, CHUNK).astype(jnp.int32)
             * T_RANGE
             + mjd.reshape(N_ROWS // CHUNK, CHUNK).astype(jnp.int32))
    out = _sc_kernel(x2, cidx2, tok_table, combo, ln_gamma, ln_beta)
    return out.reshape(B, L, D)
